# Initial kernel scaffold; baseline (speedup 1.0000x reference)
import jax
import jax.numpy as jnp
from jax.experimental import pallas as pl


def kernel(x, gw, gb, ew1, eb1, ew2, eb2, ew3, eb3, sw1, sb1, sw2, sb2, sw3, sb3):
    raise NotImplementedError("write your pallas kernel here")


# speedup vs baseline: 1.6614x; 1.6614x over previous
import functools
import jax
import jax.numpy as jnp
from jax.experimental import pallas as pl
from jax.experimental.pallas import tpu as pltpu

B, S, D = 2, 2048, 1024
E, K, I = 8, 2, 512
SH = 2 * 512
T = B * S


def _gate_body(x_ref, gw_ref, gb_ref, w_ref, idx_ref, coeff_ref):
    x = x_ref[...]
    scores = jax.lax.dot_general(
        x, gw_ref[...], (((1,), (1,)), ((), ())),
        preferred_element_type=jnp.float32) + gb_ref[...]
    m = jnp.max(scores, axis=1, keepdims=True)
    ex = jnp.exp(scores - m)
    probs = ex / jnp.sum(ex, axis=1, keepdims=True)
    w_ref[...] = probs
    iota8 = jax.lax.broadcasted_iota(jnp.int32, (T, E), 1)
    m0 = jnp.max(probs, axis=1, keepdims=True)
    i0 = jnp.min(jnp.where(probs == m0, iota8, E), axis=1)
    pm = jnp.where(iota8 == i0[:, None], -jnp.inf, probs)
    m1 = jnp.max(pm, axis=1, keepdims=True)
    i1 = jnp.min(jnp.where(pm == m1, iota8, E), axis=1)
    idx_ref[...] = jnp.concatenate([i0[:, None], i1[:, None]], axis=1)
    coeff_ref[...] = jnp.sum(probs, axis=1, keepdims=True)


def _gate(xf, gw, gb):
    return pl.pallas_call(
        _gate_body,
        out_shape=(
            jax.ShapeDtypeStruct((T, E), jnp.float32),
            jax.ShapeDtypeStruct((T, K), jnp.int32),
            jax.ShapeDtypeStruct((T, 1), jnp.float32),
        ),
    )(xf, gw, gb.reshape(1, E))


_TM = 512


def _dense_body(x_ref, idx_ref, coeff_ref, w1_ref, b1_ref, w3_ref, b3_ref,
                w2_ref, b2_ref, y_ref):
    e = pl.program_id(1)
    x = x_ref[...].astype(jnp.bfloat16)
    idx = idx_ref[...]
    mask = (idx[:, 0] == e) | (idx[:, 1] == e)
    w1 = w1_ref[0].astype(jnp.bfloat16)
    w3 = w3_ref[0].astype(jnp.bfloat16)
    w2 = w2_ref[0].astype(jnp.bfloat16)
    dims = (((1,), (1,)), ((), ()))
    h1 = jax.lax.dot_general(x, w1, dims, preferred_element_type=jnp.float32)
    h1 = h1 + b1_ref[0]
    h3 = jax.lax.dot_general(x, w3, dims, preferred_element_type=jnp.float32)
    h3 = h3 + b3_ref[0]
    h = (jax.nn.silu(h1) * h3).astype(jnp.bfloat16)
    out = jax.lax.dot_general(h, w2, dims, preferred_element_type=jnp.float32)
    out = out + b2_ref[0]
    contrib = jnp.where(mask[:, None], out * coeff_ref[...], 0.0)

    @pl.when(e == 0)
    def _():
        y_ref[...] = contrib

    @pl.when(e > 0)
    def _():
        y_ref[...] += contrib


def _dense_experts(xf, idx, coeff, ew1, eb1, ew3, eb3, ew2, eb2):
    grid = (T // _TM, E)
    return pl.pallas_call(
        _dense_body,
        grid=grid,
        in_specs=[
            pl.BlockSpec((_TM, D), lambda i, e: (i, 0)),
            pl.BlockSpec((_TM, K), lambda i, e: (i, 0)),
            pl.BlockSpec((_TM, 1), lambda i, e: (i, 0)),
            pl.BlockSpec((1, I, D), lambda i, e: (e, 0, 0)),
            pl.BlockSpec((1, 1, I), lambda i, e: (e, 0, 0)),
            pl.BlockSpec((1, I, D), lambda i, e: (e, 0, 0)),
            pl.BlockSpec((1, 1, I), lambda i, e: (e, 0, 0)),
            pl.BlockSpec((1, D, I), lambda i, e: (e, 0, 0)),
            pl.BlockSpec((1, 1, D), lambda i, e: (e, 0, 0)),
        ],
        out_specs=pl.BlockSpec((_TM, D), lambda i, e: (i, 0)),
        out_shape=jax.ShapeDtypeStruct((T, D), jnp.float32),
    )(xf, idx, coeff, ew1, eb1.reshape(E, 1, I), ew3, eb3.reshape(E, 1, I),
      ew2, eb2.reshape(E, 1, D))


def _shared_body(x_ref, y_ref, sw1_ref, sb1_ref, sw3_ref, sb3_ref, sw2_ref,
                 sb2_ref, o_ref):
    x = x_ref[...].astype(jnp.bfloat16)
    dims = (((1,), (1,)), ((), ()))
    h1 = jax.lax.dot_general(x, sw1_ref[...].astype(jnp.bfloat16), dims,
                             preferred_element_type=jnp.float32) + sb1_ref[...]
    h3 = jax.lax.dot_general(x, sw3_ref[...].astype(jnp.bfloat16), dims,
                             preferred_element_type=jnp.float32) + sb3_ref[...]
    h = (jax.nn.silu(h1) * h3).astype(jnp.bfloat16)
    z = jax.lax.dot_general(h, sw2_ref[...].astype(jnp.bfloat16), dims,
                            preferred_element_type=jnp.float32) + sb2_ref[...]
    o_ref[...] = y_ref[...] + z


def _shared_combine(xf, y, sw1, sb1, sw3, sb3, sw2, sb2):
    grid = (T // _TM,)
    return pl.pallas_call(
        _shared_body,
        grid=grid,
        in_specs=[
            pl.BlockSpec((_TM, D), lambda i: (i, 0)),
            pl.BlockSpec((_TM, D), lambda i: (i, 0)),
            pl.BlockSpec((SH, D), lambda i: (0, 0)),
            pl.BlockSpec((1, SH), lambda i: (0, 0)),
            pl.BlockSpec((SH, D), lambda i: (0, 0)),
            pl.BlockSpec((1, SH), lambda i: (0, 0)),
            pl.BlockSpec((D, SH), lambda i: (0, 0)),
            pl.BlockSpec((1, D), lambda i: (0, 0)),
        ],
        out_specs=pl.BlockSpec((_TM, D), lambda i: (i, 0)),
        out_shape=jax.ShapeDtypeStruct((T, D), jnp.float32),
    )(xf, y, sw1, sb1.reshape(1, SH), sw3, sb3.reshape(1, SH), sw2,
      sb2.reshape(1, D))


@jax.jit
def kernel(x, gw, gb, ew1, eb1, ew2, eb2, ew3, eb3, sw1, sb1, sw2, sb2, sw3,
           sb3):
    shape = x.shape
    xf = x.reshape(-1, D)
    weights, indices, coeff = _gate(xf, gw, gb)
    y = _dense_experts(xf, indices, coeff, ew1, eb1, ew3, eb3, ew2, eb2)
    out = _shared_combine(xf, y, sw1, sb1, sw3, sb3, sw2, sb2)
    return (weights, indices, out.reshape(shape))


# speedup vs baseline: 1.9142x; 1.1521x over previous
import functools
import jax
import jax.numpy as jnp
from jax import lax
from jax.experimental import pallas as pl
from jax.experimental.pallas import tpu as pltpu
from jax.experimental.pallas import tpu_sc as plsc

B, S, D = 2, 2048, 1024
E, K, I = 8, 2, 512
SH = 2 * 512
T = B * S

TM = 256
NT = T * K // TM + E
NROWS = NT * TM
NW = 32
TPW = T // NW
RC = 64
NCH = TPW // RC

_dims_nt = (((1,), (1,)), ((), ()))


def _gate_body(x_ref, gw_ref, gb_ref, w_ref, idx_ref, coeff_ref, dest_ref,
               cnt_ref, pos_ref, mask_ref):
    x = x_ref[...]
    scores = lax.dot_general(x, gw_ref[...], _dims_nt,
                             preferred_element_type=jnp.float32) + gb_ref[...]
    m = jnp.max(scores, axis=1, keepdims=True)
    ex = jnp.exp(scores - m)
    probs = ex / jnp.sum(ex, axis=1, keepdims=True)
    w_ref[...] = probs
    iota8 = lax.broadcasted_iota(jnp.int32, (T, E), 1)
    m0 = jnp.max(probs, axis=1, keepdims=True)
    i0 = jnp.min(jnp.where(probs == m0, iota8, E), axis=1)
    pm = jnp.where(iota8 == i0[:, None], -jnp.inf, probs)
    m1 = jnp.max(pm, axis=1, keepdims=True)
    i1 = jnp.min(jnp.where(pm == m1, iota8, E), axis=1)
    idx_ref[...] = jnp.concatenate([i0[:, None], i1[:, None]], axis=1)
    coeff_ref[...] = jnp.sum(probs, axis=1, keepdims=True)

    onehot0 = iota8 == i0[:, None]
    onehot1 = iota8 == i1[:, None]
    mask_ref[...] = (onehot0 | onehot1).astype(jnp.float32)
    C = 256
    r_io = lax.broadcasted_iota(jnp.int32, (C, C), 0)
    c_io = lax.broadcasted_iota(jnp.int32, (C, C), 1)
    tril = (c_io < r_io).astype(jnp.bfloat16)

    def body(c, carry):
        mc = mask_ref[pl.ds(c * C, C), :]
        posc = lax.dot_general(tril, mc.astype(jnp.bfloat16),
                               (((1,), (0,)), ((), ())),
                               preferred_element_type=jnp.float32) + carry
        pos_ref[pl.ds(c * C, C), :] = posc
        return carry + jnp.sum(mc, axis=0, keepdims=True)

    counts = lax.fori_loop(0, T // C, body, jnp.zeros((1, E), jnp.float32))
    cnt_ref[...] = counts.astype(jnp.int32)
    pc = jnp.floor((counts + (TM - 1)) / TM) * TM
    e_r = lax.broadcasted_iota(jnp.int32, (E, E), 0)
    e_c = lax.broadcasted_iota(jnp.int32, (E, E), 1)
    tril8 = (e_c < e_r).astype(jnp.float32)
    off = lax.dot_general(pc, tril8, _dims_nt,
                          preferred_element_type=jnp.float32)
    slot = off + pos_ref[...]
    d0 = jnp.sum(jnp.where(onehot0, slot, 0.0), axis=1)
    d1 = jnp.sum(jnp.where(onehot1, slot, 0.0), axis=1)
    dest_ref[...] = jnp.concatenate(
        [d0[:, None], d1[:, None]], axis=1).astype(jnp.int32)


def _gate(xf, gw, gb):
    return pl.pallas_call(
        _gate_body,
        out_shape=(
            jax.ShapeDtypeStruct((T, E), jnp.float32),
            jax.ShapeDtypeStruct((T, K), jnp.int32),
            jax.ShapeDtypeStruct((T, 1), jnp.float32),
            jax.ShapeDtypeStruct((T, K), jnp.int32),
            jax.ShapeDtypeStruct((1, E), jnp.int32),
        ),
        scratch_shapes=[pltpu.VMEM((T, E), jnp.float32),
                        pltpu.VMEM((T, E), jnp.float32)],
    )(xf, gw, gb.reshape(1, E))


def _sc_dispatch(xf, d3):
    mesh = plsc.VectorSubcoreMesh(core_axis_name="c", subcore_axis_name="s")

    @functools.partial(
        pl.kernel, mesh=mesh,
        out_type=jax.ShapeDtypeStruct((NROWS, D), jnp.float32),
        scratch_types=[pltpu.VMEM((K * NCH, RC), jnp.int32),
                       pltpu.VMEM((RC, D), jnp.float32)],
    )
    def run(x_hbm, d3_hbm, xs_hbm, idx_v, rows_v):
        wid = lax.axis_index("s") * 2 + lax.axis_index("c")
        pltpu.sync_copy(d3_hbm.at[wid], idx_v)

        @pl.loop(0, NCH)
        def _(c):
            base = wid * TPW + c * RC
            pltpu.sync_copy(x_hbm.at[pl.ds(base, RC)], rows_v)
            pltpu.sync_copy(rows_v, xs_hbm.at[idx_v.at[c]])
            pltpu.sync_copy(rows_v, xs_hbm.at[idx_v.at[NCH + c]])

    return run(xf, d3)


def _sc_combine(o_rows, d3):
    mesh = plsc.VectorSubcoreMesh(core_axis_name="c", subcore_axis_name="s")

    @functools.partial(
        pl.kernel, mesh=mesh,
        out_type=(jax.ShapeDtypeStruct((T, D), jnp.float32),
                  jax.ShapeDtypeStruct((T, D), jnp.float32)),
        scratch_types=[pltpu.VMEM((K * NCH, RC), jnp.int32),
                       pltpu.VMEM((RC, D), jnp.float32)],
    )
    def run(o_hbm, d3_hbm, y0_hbm, y1_hbm, idx_v, rows_v):
        wid = lax.axis_index("s") * 2 + lax.axis_index("c")
        pltpu.sync_copy(d3_hbm.at[wid], idx_v)

        @pl.loop(0, NCH)
        def _(c):
            base = wid * TPW + c * RC
            pltpu.sync_copy(o_hbm.at[idx_v.at[c]], rows_v)
            pltpu.sync_copy(rows_v, y0_hbm.at[pl.ds(base, RC)])
            pltpu.sync_copy(o_hbm.at[idx_v.at[NCH + c]], rows_v)
            pltpu.sync_copy(rows_v, y1_hbm.at[pl.ds(base, RC)])

    return run(o_rows, d3)


def _group_body(meta_ref, xs_ref, w1_ref, b1_ref, w3_ref, b3_ref, w2_ref,
                b2_ref, o_ref):
    x = xs_ref[...].astype(jnp.bfloat16)
    h1 = lax.dot_general(x, w1_ref[0].astype(jnp.bfloat16), _dims_nt,
                         preferred_element_type=jnp.float32) + b1_ref[0]
    h3 = lax.dot_general(x, w3_ref[0].astype(jnp.bfloat16), _dims_nt,
                         preferred_element_type=jnp.float32) + b3_ref[0]
    h = (jax.nn.silu(h1) * h3).astype(jnp.bfloat16)
    o_ref[...] = lax.dot_general(h, w2_ref[0].astype(jnp.bfloat16), _dims_nt,
                                 preferred_element_type=jnp.float32) + b2_ref[0]


def _grouped_matmul(tile_expert, xs, ew1, eb1, ew3, eb3, ew2, eb2):
    grid_spec = pltpu.PrefetchScalarGridSpec(
        num_scalar_prefetch=1,
        grid=(NT,),
        in_specs=[
            pl.BlockSpec((TM, D), lambda i, m: (i, 0)),
            pl.BlockSpec((1, I, D), lambda i, m: (m[i], 0, 0)),
            pl.BlockSpec((1, 1, I), lambda i, m: (m[i], 0, 0)),
            pl.BlockSpec((1, I, D), lambda i, m: (m[i], 0, 0)),
            pl.BlockSpec((1, 1, I), lambda i, m: (m[i], 0, 0)),
            pl.BlockSpec((1, D, I), lambda i, m: (m[i], 0, 0)),
            pl.BlockSpec((1, 1, D), lambda i, m: (m[i], 0, 0)),
        ],
        out_specs=pl.BlockSpec((TM, D), lambda i, m: (i, 0)),
    )
    return pl.pallas_call(
        _group_body,
        grid_spec=grid_spec,
        out_shape=jax.ShapeDtypeStruct((NROWS, D), jnp.float32),
    )(tile_expert, xs, ew1, eb1.reshape(E, 1, I), ew3, eb3.reshape(E, 1, I),
      ew2, eb2.reshape(E, 1, D))


_TMS = 512


def _shared_body(x_ref, sw1_ref, sb1_ref, sw3_ref, sb3_ref, sw2_ref, sb2_ref,
                 z_ref):
    x = x_ref[...].astype(jnp.bfloat16)
    h1 = lax.dot_general(x, sw1_ref[...].astype(jnp.bfloat16), _dims_nt,
                         preferred_element_type=jnp.float32) + sb1_ref[...]
    h3 = lax.dot_general(x, sw3_ref[...].astype(jnp.bfloat16), _dims_nt,
                         preferred_element_type=jnp.float32) + sb3_ref[...]
    h = (jax.nn.silu(h1) * h3).astype(jnp.bfloat16)
    z_ref[...] = lax.dot_general(h, sw2_ref[...].astype(jnp.bfloat16),
                                 _dims_nt,
                                 preferred_element_type=jnp.float32) + sb2_ref[...]


def _shared(xf, sw1, sb1, sw3, sb3, sw2, sb2):
    return pl.pallas_call(
        _shared_body,
        grid=(T // _TMS,),
        in_specs=[
            pl.BlockSpec((_TMS, D), lambda i: (i, 0)),
            pl.BlockSpec((SH, D), lambda i: (0, 0)),
            pl.BlockSpec((1, SH), lambda i: (0, 0)),
            pl.BlockSpec((SH, D), lambda i: (0, 0)),
            pl.BlockSpec((1, SH), lambda i: (0, 0)),
            pl.BlockSpec((D, SH), lambda i: (0, 0)),
            pl.BlockSpec((1, D), lambda i: (0, 0)),
        ],
        out_specs=pl.BlockSpec((_TMS, D), lambda i: (i, 0)),
        out_shape=jax.ShapeDtypeStruct((T, D), jnp.float32),
    )(xf, sw1, sb1.reshape(1, SH), sw3, sb3.reshape(1, SH), sw2,
      sb2.reshape(1, D))


def _epilogue_body(y0_ref, y1_ref, coeff_ref, z_ref, o_ref):
    o_ref[...] = (y0_ref[...] + y1_ref[...]) * coeff_ref[...] + z_ref[...]


def _epilogue(y0, y1, coeff, z):
    return pl.pallas_call(
        _epilogue_body,
        grid=(T // _TMS,),
        in_specs=[
            pl.BlockSpec((_TMS, D), lambda i: (i, 0)),
            pl.BlockSpec((_TMS, D), lambda i: (i, 0)),
            pl.BlockSpec((_TMS, 1), lambda i: (i, 0)),
            pl.BlockSpec((_TMS, D), lambda i: (i, 0)),
        ],
        out_specs=pl.BlockSpec((_TMS, D), lambda i: (i, 0)),
        out_shape=jax.ShapeDtypeStruct((T, D), jnp.float32),
    )(y0, y1, coeff, z)


@jax.jit
def kernel(x, gw, gb, ew1, eb1, ew2, eb2, ew3, eb3, sw1, sb1, sw2, sb2, sw3,
           sb3):
    shape = x.shape
    xf = x.reshape(-1, D)
    weights, indices, coeff, dest, cnt = _gate(xf, gw, gb)

    counts = cnt[0]
    pcnt = ((counts + TM - 1) // TM) * TM
    ends = jnp.cumsum(pcnt) // TM
    tix = jnp.arange(NT, dtype=jnp.int32)
    tile_expert = jnp.minimum(
        jnp.sum((tix[:, None] >= ends[None, :]).astype(jnp.int32), axis=1),
        E - 1).astype(jnp.int32)

    d3 = dest.reshape(NW, NCH, RC, K).transpose(0, 3, 1, 2).reshape(
        NW, K * NCH, RC)

    xs = _sc_dispatch(xf, d3)
    z = _shared(xf, sw1, sb1, sw3, sb3, sw2, sb2)
    o_rows = _grouped_matmul(tile_expert, xs, ew1, eb1, ew3, eb3, ew2, eb2)
    y0, y1 = _sc_combine(o_rows, d3)
    out = _epilogue(y0, y1, coeff, z)
    return (weights, indices, out.reshape(shape))
